# baseline (speedup 1.0000x reference)
import functools

import jax
import jax.numpy as jnp
from jax import lax
from jax.experimental import pallas as pl
from jax.experimental.pallas import tpu as pltpu
from jax.experimental.pallas import tpu_sc as plsc

_NW = 32
_CHUNK = 16384
_LANES = 16


def _sc_body(x_hbm, pos_hbm, out_hbm, xbuf, pbuf, obuf, *, per_w, pos_words):
    wid = lax.axis_index("s") * 2 + lax.axis_index("c")
    xoff = pl.multiple_of(wid * per_w, _CHUNK)
    poff = pl.multiple_of((wid * per_w) % pos_words, _CHUNK)
    n_chunks = per_w // _CHUNK

    def chunk_body(k, carry):
        base = pl.multiple_of(xoff + k * _CHUNK, _CHUNK)
        pbase = pl.multiple_of(poff + k * _CHUNK, _CHUNK)
        pltpu.sync_copy(x_hbm.at[pl.ds(base, _CHUNK)], xbuf)
        pltpu.sync_copy(pos_hbm.at[pl.ds(pbase, _CHUNK)], pbuf)

        def add_body(j, c):
            sl = pl.ds(j * _LANES, _LANES)
            obuf[sl] = xbuf[sl] + pbuf[sl]
            return c

        lax.fori_loop(0, _CHUNK // _LANES, add_body, 0)
        pltpu.sync_copy(obuf, out_hbm.at[pl.ds(base, _CHUNK)])
        return carry

    lax.fori_loop(0, n_chunks, chunk_body, 0)


def kernel(x, position_embeddings):
    B, T, C = x.shape
    pos = position_embeddings[:T]
    x_words = B * T * C
    pos_words = T * C
    per_w = x_words // _NW

    mesh = plsc.VectorSubcoreMesh(core_axis_name="c", subcore_axis_name="s")
    sc_call = pl.kernel(
        functools.partial(_sc_body, per_w=per_w, pos_words=pos_words),
        mesh=mesh,
        out_type=jax.ShapeDtypeStruct((x_words,), jnp.float32),
        scratch_types=[
            pltpu.VMEM((_CHUNK,), jnp.float32),
            pltpu.VMEM((_CHUNK,), jnp.float32),
            pltpu.VMEM((_CHUNK,), jnp.float32),
        ],
    )
    out = sc_call(x.reshape(-1), pos.reshape(-1))
    return out.reshape(B, T, C)


# speedup vs baseline: 7.8448x; 7.8448x over previous
import jax
import jax.numpy as jnp
from jax.experimental import pallas as pl


_ROWS = 1024


def _add_kernel(x_ref, pos_ref, out_ref):
    out_ref[...] = x_ref[...] + pos_ref[...]


def kernel(x, position_embeddings):
    B, T, C = x.shape
    pos = position_embeddings[:T]
    xf = x.reshape(B * T, C)
    nt = T // _ROWS
    out = pl.pallas_call(
        _add_kernel,
        grid=(nt, B),
        in_specs=[
            pl.BlockSpec((_ROWS, C), lambda t, b: (b * nt + t, 0)),
            pl.BlockSpec((_ROWS, C), lambda t, b: (t, 0)),
        ],
        out_specs=pl.BlockSpec((_ROWS, C), lambda t, b: (b * nt + t, 0)),
        out_shape=jax.ShapeDtypeStruct((B * T, C), x.dtype),
    )(xf, pos)
    return out.reshape(B, T, C)
